# baseline (speedup 1.0000x reference)
import functools

import jax
import jax.numpy as jnp
from jax import lax
from jax.experimental import pallas as pl
from jax.experimental.pallas import tpu as pltpu
from jax.experimental.pallas import tpu_sc as plsc

_N = 16
_D = 16
_K = 8
_BB = 1024


def _lane_consts():
    l = jax.lax.broadcasted_iota(jnp.int32, (1, _N * _D), 1)
    return l // _D, l % _D


def _stage_a_body(x_ref, ew1, eb1, ew2, eb2, gw1, gb1, gw2, gb2,
                  khot_ref, fsel_ref):
    f32 = jnp.float32
    dot = lambda a, b: jax.lax.dot_general(
        a.astype(jnp.bfloat16), b.astype(jnp.bfloat16),
        (((1,), (1,)), ((), ())), preferred_element_type=f32)
    mml = lambda a, b: jax.lax.dot_general(
        a, b, (((1,), (0,)), ((), ())), preferred_element_type=f32)
    mmh = lambda a, b: jax.lax.dot_general(
        a, b, (((1,), (0,)), ((), ())), preferred_element_type=f32,
        precision=jax.lax.Precision.HIGHEST)

    x = x_ref[...]
    h1 = dot(x, ew1[...]) + eb1[...]
    h1 = h1 * jax.nn.sigmoid(h1)
    enc = dot(h1, ew2[...]) + eb2[...]
    g1 = dot(enc, gw1[...]) + gb1[...]
    g1 = g1 * jax.nn.sigmoid(g1)
    gate = dot(g1, gw2[...]) + gb2[...]

    seg_i, seg_j = _lane_consts()
    pi = jax.lax.broadcasted_iota(jnp.int32, (_N, _N * _D), 0)
    R_rep = (pi == seg_i).astype(f32)
    Mseg = (seg_i.T == jax.lax.broadcasted_iota(
        jnp.int32, (_N * _D, _N), 1)).astype(f32)

    gate_t = pltpu.repeat(gate, _N, 1)
    gate_r = mmh(gate, R_rep)
    tie = (seg_j < seg_i)
    beats = jnp.where((gate_t > gate_r) | ((gate_t == gate_r) & tie), 1.0, 0.0)
    rank16 = mml(beats, Mseg)

    m = jnp.max(enc, axis=1, keepdims=True)
    e = jnp.exp(enc - m)
    seg_sum = mmh(e, Mseg)
    sum_lane = mmh(seg_sum, R_rep)
    khot_ref[...] = jnp.where(e * _D > sum_lane, 1.0, 0.0)

    lane128 = jax.lax.broadcasted_iota(jnp.int32, (1, _K * _N), 1)
    rconst = lane128 // _N
    iconst = (lane128 % _N).astype(f32)
    M128 = ((lane128.T // _N) == jax.lax.broadcasted_iota(
        jnp.int32, (_K * _N, _K), 1)).astype(f32)
    rank_rep = pltpu.repeat(rank16, _K, 1)
    selv = jnp.where(rank_rep == rconst.astype(f32), iconst, 0.0)
    sel8 = mml(selv, M128)
    row = (pl.program_id(0) * _BB
           + jax.lax.broadcasted_iota(jnp.int32, (_BB, 1), 0))
    fsel_ref[...] = (sel8 + 16.0 * row.astype(f32)).astype(jnp.int32)


def _stage_b_body(hot_ref, cbw, dw1, db1, dw2, db2, recon_ref, kk_ref):
    f32 = jnp.float32
    dot = lambda a, b: jax.lax.dot_general(
        a.astype(jnp.bfloat16), b.astype(jnp.bfloat16),
        (((1,), (1,)), ((), ())), preferred_element_type=f32)
    hot = hot_ref[...]
    kk = jnp.clip(jnp.sum(hot, axis=1, keepdims=True), 1.0, 128.0)
    hot_n = hot / kk
    q = dot(hot_n, cbw[...])
    d1 = dot(q, dw1[...]) + db1[...]
    d1 = d1 * jax.nn.sigmoid(d1)
    recon_ref[...] = dot(d1, dw2[...]) + db2[...]
    kk_ref[...] = kk


def _sc_gather(table, idx2d, rows_total, b_per_w, nch, ch):
    mesh = plsc.VectorSubcoreMesh(core_axis_name="c", subcore_axis_name="s")

    @functools.partial(
        pl.kernel, mesh=mesh,
        compiler_params=pltpu.CompilerParams(use_tc_tiling_on_sc=False),
        out_type=jax.ShapeDtypeStruct((rows_total, _D), jnp.float32),
        scratch_types=[
            pltpu.VMEM((nch, ch), jnp.int32),
            pltpu.VMEM((b_per_w, _D), jnp.float32),
            pltpu.SemaphoreType.DMA,
        ],
    )
    def k(table_hbm, idx_hbm, out_hbm, idx_v, rows_v, sem):
        wid = lax.axis_index("s") * 2 + lax.axis_index("c")
        pltpu.sync_copy(idx_hbm.at[pl.ds(wid * nch, nch)], idx_v)
        copies = []
        for j in range(nch):
            copies.append(pltpu.async_copy(
                table_hbm.at[idx_v.at[j]],
                rows_v.at[pl.ds(j * ch, ch)], sem))
        for c in copies:
            c.wait()
        pltpu.sync_copy(rows_v, out_hbm.at[pl.ds(wid * b_per_w, b_per_w)])

    return k(table, idx2d)


def _chunk(x, ew1, eb1, ew2, eb2, gw1, gb1, gw2, gb2):
    b = x.shape[0]
    grid = (b // _BB,)
    full = lambda shape: pl.BlockSpec(shape, lambda i: (0,) * len(shape))
    khot, fsel = pl.pallas_call(
        _stage_a_body,
        grid=grid,
        in_specs=[
            pl.BlockSpec((_BB, 128), lambda i: (i, 0)),
            full((256, 128)), full((1, 256)),
            full((256, 256)), full((1, 256)),
            full((256, 256)), full((1, 256)),
            full((16, 256)), full((1, 16)),
        ],
        out_specs=[
            pl.BlockSpec((_BB, 256), lambda i: (i, 0)),
            pl.BlockSpec((_BB, _K), lambda i: (i, 0)),
        ],
        out_shape=[
            jax.ShapeDtypeStruct((b, 256), jnp.float32),
            jax.ShapeDtypeStruct((b, _K), jnp.int32),
        ],
    )(x, ew1, eb1.reshape(1, -1), ew2, eb2.reshape(1, -1), gw1,
      gb1.reshape(1, -1), gw2, gb2.reshape(1, -1))

    info = plsc.get_sparse_core_info()
    nw = info.num_cores * info.num_subcores
    rows_total = b * _K
    b_per_w = rows_total // nw
    ch = 128
    nch = b_per_w // ch
    table = khot.reshape(b * _N, _D)
    idx2d = fsel.reshape(rows_total // ch, ch)
    hot_rows = _sc_gather(table, idx2d, rows_total, b_per_w, nch, ch)
    return hot_rows.reshape(b, _K * _D)


def _decode(hot, cbw, dw1, db1, dw2, db2):
    b = hot.shape[0]
    grid = (b // _BB,)
    full = lambda shape: pl.BlockSpec(shape, lambda i: (0,) * len(shape))
    recon, kk = pl.pallas_call(
        _stage_b_body,
        grid=grid,
        in_specs=[
            pl.BlockSpec((_BB, 128), lambda i: (i, 0)),
            full((64, 128)),
            full((256, 64)), full((1, 256)),
            full((128, 256)), full((1, 128)),
        ],
        out_specs=[
            pl.BlockSpec((_BB, 128), lambda i: (i, 0)),
            pl.BlockSpec((_BB, 1), lambda i: (i, 0)),
        ],
        out_shape=[
            jax.ShapeDtypeStruct((b, 128), jnp.float32),
            jax.ShapeDtypeStruct((b, 1), jnp.float32),
        ],
    )(hot, cbw, dw1, db1.reshape(1, -1), dw2, db2.reshape(1, -1))
    return recon, kk


_NCHUNK = 2


@jax.jit
def _run(x, ew1, eb1, ew2, eb2, gw1, gb1, gw2, gb2, cbw, dw1, db1, dw2, db2):
    b = x.shape[0]
    cs = b // _NCHUNK
    hots = [_chunk(x[c * cs:(c + 1) * cs], ew1, eb1, ew2, eb2,
                   gw1, gb1, gw2, gb2) for c in range(_NCHUNK)]
    outs = [_decode(h, cbw, dw1, db1, dw2, db2) for h in hots]
    recon = jnp.concatenate([o[0] for o in outs])
    kk = jnp.concatenate([o[1] for o in outs])
    hot = jnp.concatenate(hots)
    return recon, hot, kk


def kernel(x, enc_w1, enc_b1, enc_w2, enc_b2, gate_w1, gate_b1, gate_w2,
           gate_b2, cb_w, dec_w1, dec_b1, dec_w2, dec_b2):
    recon, hot, kk = _run(x, enc_w1, enc_b1, enc_w2, enc_b2, gate_w1, gate_b1,
                          gate_w2, gate_b2, cb_w, dec_w1, dec_b1, dec_w2,
                          dec_b2)
    return (recon, hot, jnp.float32(0.0), kk)


# speedup vs baseline: 1.1772x; 1.1772x over previous
import functools

import jax
import jax.numpy as jnp
from jax import lax
from jax.experimental import pallas as pl
from jax.experimental.pallas import tpu as pltpu
from jax.experimental.pallas import tpu_sc as plsc

_N = 16
_D = 16
_K = 8
_BB = 1024


def _lane_consts():
    l = jax.lax.broadcasted_iota(jnp.int32, (1, _N * _D), 1)
    return l // _D, l % _D


def _stage_a_body(x_ref, ew1, eb1, ew2, eb2, gw1, gb1, gw2, gb2,
                  khot_ref, fsel_ref):
    f32 = jnp.float32
    dot = lambda a, b: jax.lax.dot_general(
        a.astype(jnp.bfloat16), b.astype(jnp.bfloat16),
        (((1,), (1,)), ((), ())), preferred_element_type=f32)
    mml = lambda a, b: jax.lax.dot_general(
        a, b, (((1,), (0,)), ((), ())), preferred_element_type=f32)
    mmh = lambda a, b: jax.lax.dot_general(
        a, b, (((1,), (0,)), ((), ())), preferred_element_type=f32,
        precision=jax.lax.Precision.HIGHEST)

    x = x_ref[...]
    h1 = dot(x, ew1[...]) + eb1[...]
    h1 = h1 * jax.nn.sigmoid(h1)
    enc = dot(h1, ew2[...]) + eb2[...]
    g1 = dot(enc, gw1[...]) + gb1[...]
    g1 = g1 * jax.nn.sigmoid(g1)
    gate = dot(g1, gw2[...]) + gb2[...]

    seg_i, seg_j = _lane_consts()
    pi = jax.lax.broadcasted_iota(jnp.int32, (_N, _N * _D), 0)
    R_rep = (pi == seg_i).astype(f32)
    Mseg = (seg_i.T == jax.lax.broadcasted_iota(
        jnp.int32, (_N * _D, _N), 1)).astype(f32)

    gate_t = pltpu.repeat(gate, _N, 1)
    gate_r = mmh(gate, R_rep)
    tie = (seg_j < seg_i)
    beats = jnp.where((gate_t > gate_r) | ((gate_t == gate_r) & tie), 1.0, 0.0)
    rank16 = mml(beats, Mseg)

    S = (seg_i.T == seg_i).astype(f32)
    m = jnp.max(enc, axis=1, keepdims=True)
    e = jnp.exp(enc - m)
    sum_lane = mmh(e, S)
    khot_ref[...] = jnp.where(e * _D > sum_lane, 1.0, 0.0)

    lane128 = jax.lax.broadcasted_iota(jnp.int32, (1, _K * _N), 1)
    rconst = lane128 // _N
    iconst = (lane128 % _N).astype(f32)
    M128 = ((lane128.T // _N) == jax.lax.broadcasted_iota(
        jnp.int32, (_K * _N, _K), 1)).astype(f32)
    rank_rep = pltpu.repeat(rank16, _K, 1)
    selv = jnp.where(rank_rep == rconst.astype(f32), iconst, 0.0)
    sel8 = mml(selv, M128)
    row = (pl.program_id(0) * _BB
           + jax.lax.broadcasted_iota(jnp.int32, (_BB, 1), 0))
    fsel_ref[...] = (sel8 + 16.0 * row.astype(f32)).astype(jnp.int32)


def _stage_b_body(hot_ref, cbw, dw1, db1, dw2, db2, recon_ref, kk_ref):
    f32 = jnp.float32
    dot = lambda a, b: jax.lax.dot_general(
        a.astype(jnp.bfloat16), b.astype(jnp.bfloat16),
        (((1,), (1,)), ((), ())), preferred_element_type=f32)
    hot = hot_ref[...]
    kk = jnp.clip(jnp.sum(hot, axis=1, keepdims=True), 1.0, 128.0)
    hot_n = hot / kk
    q = dot(hot_n, cbw[...])
    d1 = dot(q, dw1[...]) + db1[...]
    d1 = d1 * jax.nn.sigmoid(d1)
    recon_ref[...] = dot(d1, dw2[...]) + db2[...]
    kk_ref[...] = kk


def _sc_gather(table, idx2d, rows_total, b_per_w, nch, ch):
    mesh = plsc.VectorSubcoreMesh(core_axis_name="c", subcore_axis_name="s")

    @functools.partial(
        pl.kernel, mesh=mesh,
        compiler_params=pltpu.CompilerParams(use_tc_tiling_on_sc=False),
        out_type=jax.ShapeDtypeStruct((rows_total, _D), jnp.float32),
        scratch_types=[
            pltpu.VMEM((nch, ch), jnp.int32),
            pltpu.VMEM((b_per_w, _D), jnp.float32),
            pltpu.SemaphoreType.DMA,
        ],
    )
    def k(table_hbm, idx_hbm, out_hbm, idx_v, rows_v, sem):
        wid = lax.axis_index("s") * 2 + lax.axis_index("c")
        pltpu.sync_copy(idx_hbm.at[pl.ds(wid * nch, nch)], idx_v)
        copies = []
        for j in range(nch):
            copies.append(pltpu.async_copy(
                table_hbm.at[idx_v.at[j]],
                rows_v.at[pl.ds(j * ch, ch)], sem))
        for c in copies:
            c.wait()
        pltpu.sync_copy(rows_v, out_hbm.at[pl.ds(wid * b_per_w, b_per_w)])

    return k(table, idx2d)


def _chunk(x, ew1, eb1, ew2, eb2, gw1, gb1, gw2, gb2):
    b = x.shape[0]
    grid = (b // _BB,)
    full = lambda shape: pl.BlockSpec(shape, lambda i: (0,) * len(shape))
    khot, fsel = pl.pallas_call(
        _stage_a_body,
        grid=grid,
        in_specs=[
            pl.BlockSpec((_BB, 128), lambda i: (i, 0)),
            full((256, 128)), full((1, 256)),
            full((256, 256)), full((1, 256)),
            full((256, 256)), full((1, 256)),
            full((16, 256)), full((1, 16)),
        ],
        out_specs=[
            pl.BlockSpec((_BB, 256), lambda i: (i, 0)),
            pl.BlockSpec((_BB, _K), lambda i: (i, 0)),
        ],
        out_shape=[
            jax.ShapeDtypeStruct((b, 256), jnp.float32),
            jax.ShapeDtypeStruct((b, _K), jnp.int32),
        ],
    )(x, ew1, eb1.reshape(1, -1), ew2, eb2.reshape(1, -1), gw1,
      gb1.reshape(1, -1), gw2, gb2.reshape(1, -1))

    info = plsc.get_sparse_core_info()
    nw = info.num_cores * info.num_subcores
    rows_total = b * _K
    b_per_w = rows_total // nw
    ch = 128
    nch = b_per_w // ch
    table = khot.reshape(b * _N, _D)
    idx2d = fsel.reshape(rows_total // ch, ch)
    hot_rows = _sc_gather(table, idx2d, rows_total, b_per_w, nch, ch)
    return hot_rows.reshape(b, _K * _D)


def _decode(hot, cbw, dw1, db1, dw2, db2):
    b = hot.shape[0]
    grid = (b // _BB,)
    full = lambda shape: pl.BlockSpec(shape, lambda i: (0,) * len(shape))
    recon, kk = pl.pallas_call(
        _stage_b_body,
        grid=grid,
        in_specs=[
            pl.BlockSpec((_BB, 128), lambda i: (i, 0)),
            full((64, 128)),
            full((256, 64)), full((1, 256)),
            full((128, 256)), full((1, 128)),
        ],
        out_specs=[
            pl.BlockSpec((_BB, 128), lambda i: (i, 0)),
            pl.BlockSpec((_BB, 1), lambda i: (i, 0)),
        ],
        out_shape=[
            jax.ShapeDtypeStruct((b, 128), jnp.float32),
            jax.ShapeDtypeStruct((b, 1), jnp.float32),
        ],
    )(hot, cbw, dw1, db1.reshape(1, -1), dw2, db2.reshape(1, -1))
    return recon, kk


_SPLITS = (12288, 4096)


@jax.jit
def _run(x, ew1, eb1, ew2, eb2, gw1, gb1, gw2, gb2, cbw, dw1, db1, dw2, db2):
    hots, off = [], 0
    for cs in _SPLITS:
        hots.append(_chunk(x[off:off + cs], ew1, eb1, ew2, eb2,
                           gw1, gb1, gw2, gb2))
        off += cs
    outs = [_decode(h, cbw, dw1, db1, dw2, db2) for h in hots]
    recon = jnp.concatenate([o[0] for o in outs])
    kk = jnp.concatenate([o[1] for o in outs])
    hot = jnp.concatenate(hots)
    return recon, hot, kk


def kernel(x, enc_w1, enc_b1, enc_w2, enc_b2, gate_w1, gate_b1, gate_w2,
           gate_b2, cb_w, dec_w1, dec_b1, dec_w2, dec_b2):
    recon, hot, kk = _run(x, enc_w1, enc_b1, enc_w2, enc_b2, gate_w1, gate_b1,
                          gate_w2, gate_b2, cb_w, dec_w1, dec_b1, dec_w2,
                          dec_b2)
    return (recon, hot, jnp.float32(0.0), kk)


# speedup vs baseline: 1.2395x; 1.0528x over previous
import functools

import jax
import jax.numpy as jnp
from jax import lax
from jax.experimental import pallas as pl
from jax.experimental.pallas import tpu as pltpu
from jax.experimental.pallas import tpu_sc as plsc

_N = 16
_D = 16
_K = 8
_BB = 1024


def _lane_consts():
    l = jax.lax.broadcasted_iota(jnp.int32, (1, _N * _D), 1)
    return l // _D, l % _D


def _stage_a_body(x_ref, ew1, eb1, ew2, eb2, gw1, gb1, gw2, gb2,
                  khot_ref, fsel_ref):
    f32 = jnp.float32
    dot = lambda a, b: jax.lax.dot_general(
        a.astype(jnp.bfloat16), b.astype(jnp.bfloat16),
        (((1,), (1,)), ((), ())), preferred_element_type=f32)
    mml = lambda a, b: jax.lax.dot_general(
        a, b, (((1,), (0,)), ((), ())), preferred_element_type=f32)
    mmh = lambda a, b: jax.lax.dot_general(
        a, b, (((1,), (0,)), ((), ())), preferred_element_type=f32,
        precision=jax.lax.Precision.HIGHEST)

    x = x_ref[...]
    h1 = dot(x, ew1[...]) + eb1[...]
    h1 = h1 * jax.nn.sigmoid(h1)
    enc = dot(h1, ew2[...]) + eb2[...]
    g1 = dot(enc, gw1[...]) + gb1[...]
    g1 = g1 * jax.nn.sigmoid(g1)
    gate = dot(g1, gw2[...]) + gb2[...]

    seg_i, seg_j = _lane_consts()
    pi = jax.lax.broadcasted_iota(jnp.int32, (_N, _N * _D), 0)
    R_rep = (pi == seg_i).astype(f32)
    Mseg = (seg_i.T == jax.lax.broadcasted_iota(
        jnp.int32, (_N * _D, _N), 1)).astype(f32)

    gate_t = pltpu.repeat(gate, _N, 1)
    gate_r = mmh(gate, R_rep)
    tie = (seg_j < seg_i)
    beats = jnp.where((gate_t > gate_r) | ((gate_t == gate_r) & tie), 1.0, 0.0)
    rank16 = mml(beats, Mseg)

    S = (seg_i.T == seg_i).astype(f32)
    m = jnp.max(enc, axis=1, keepdims=True)
    e = jnp.exp(enc - m)
    sum_lane = mmh(e, S)
    khot_ref[...] = jnp.where(e * _D > sum_lane, 1.0, 0.0)

    lane128 = jax.lax.broadcasted_iota(jnp.int32, (1, _K * _N), 1)
    rconst = lane128 // _N
    iconst = (lane128 % _N).astype(f32)
    M128 = ((lane128.T // _N) == jax.lax.broadcasted_iota(
        jnp.int32, (_K * _N, _K), 1)).astype(f32)
    rank_rep = pltpu.repeat(rank16, _K, 1)
    selv = jnp.where(rank_rep == rconst.astype(f32), iconst, 0.0)
    sel8 = mml(selv, M128)
    row = (pl.program_id(0) * _BB
           + jax.lax.broadcasted_iota(jnp.int32, (_BB, 1), 0))
    fsel_ref[...] = (sel8 + 16.0 * row.astype(f32)).astype(jnp.int32)


def _stage_b_body(hot0_ref, hot1_ref, cbw, dw1, db1, dw2, db2,
                  recon_ref, kk_ref):
    f32 = jnp.float32
    dot = lambda a, b: jax.lax.dot_general(
        a.astype(jnp.bfloat16), b.astype(jnp.bfloat16),
        (((1,), (1,)), ((), ())), preferred_element_type=f32)
    n0 = _SPLITS[0] // _BB
    hot = jnp.where(pl.program_id(0) < n0, hot0_ref[...], hot1_ref[...])
    kk = jnp.clip(jnp.sum(hot, axis=1, keepdims=True), 1.0, 128.0)
    hot_n = hot / kk
    q = dot(hot_n, cbw[...])
    d1 = dot(q, dw1[...]) + db1[...]
    d1 = d1 * jax.nn.sigmoid(d1)
    recon_ref[...] = dot(d1, dw2[...]) + db2[...]
    kk_ref[...] = kk


def _sc_gather(table, idx2d, rows_total, b_per_w, nch, ch):
    mesh = plsc.VectorSubcoreMesh(core_axis_name="c", subcore_axis_name="s")

    @functools.partial(
        pl.kernel, mesh=mesh,
        compiler_params=pltpu.CompilerParams(use_tc_tiling_on_sc=False),
        out_type=jax.ShapeDtypeStruct((rows_total, _D), jnp.float32),
        scratch_types=[
            pltpu.VMEM((nch, ch), jnp.int32),
            pltpu.VMEM((b_per_w, _D), jnp.float32),
            pltpu.SemaphoreType.DMA,
        ],
    )
    def k(table_hbm, idx_hbm, out_hbm, idx_v, rows_v, sem):
        wid = lax.axis_index("s") * 2 + lax.axis_index("c")
        pltpu.sync_copy(idx_hbm.at[pl.ds(wid * nch, nch)], idx_v)
        copies = []
        for j in range(nch):
            copies.append(pltpu.async_copy(
                table_hbm.at[idx_v.at[j]],
                rows_v.at[pl.ds(j * ch, ch)], sem))
        for c in copies:
            c.wait()
        pltpu.sync_copy(rows_v, out_hbm.at[pl.ds(wid * b_per_w, b_per_w)])

    return k(table, idx2d)


def _chunk(x, off, b, ew1, eb1, ew2, eb2, gw1, gb1, gw2, gb2):
    grid = (b // _BB,)
    offb = off // _BB
    full = lambda shape: pl.BlockSpec(shape, lambda i: (0,) * len(shape))
    khot, fsel = pl.pallas_call(
        _stage_a_body,
        grid=grid,
        in_specs=[
            pl.BlockSpec((_BB, 128), lambda i: (i + offb, 0)),
            full((256, 128)), full((1, 256)),
            full((256, 256)), full((1, 256)),
            full((256, 256)), full((1, 256)),
            full((16, 256)), full((1, 16)),
        ],
        out_specs=[
            pl.BlockSpec((_BB, 256), lambda i: (i, 0)),
            pl.BlockSpec((_BB, _K), lambda i: (i, 0)),
        ],
        out_shape=[
            jax.ShapeDtypeStruct((b, 256), jnp.float32),
            jax.ShapeDtypeStruct((b, _K), jnp.int32),
        ],
    )(x, ew1, eb1.reshape(1, -1), ew2, eb2.reshape(1, -1), gw1,
      gb1.reshape(1, -1), gw2, gb2.reshape(1, -1))

    info = plsc.get_sparse_core_info()
    nw = info.num_cores * info.num_subcores
    rows_total = b * _K
    b_per_w = rows_total // nw
    ch = 128
    nch = b_per_w // ch
    table = khot.reshape(b * _N, _D)
    idx2d = fsel.reshape(rows_total // ch, ch)
    hot_rows = _sc_gather(table, idx2d, rows_total, b_per_w, nch, ch)
    return hot_rows.reshape(b, _K * _D)


def _decode(hot0, hot1, cbw, dw1, db1, dw2, db2):
    b = hot0.shape[0] + hot1.shape[0]
    grid = (b // _BB,)
    n0 = hot0.shape[0] // _BB
    full = lambda shape: pl.BlockSpec(shape, lambda i: (0,) * len(shape))
    recon, kk = pl.pallas_call(
        _stage_b_body,
        grid=grid,
        in_specs=[
            pl.BlockSpec((_BB, 128), lambda i: (jnp.minimum(i, n0 - 1), 0)),
            pl.BlockSpec((_BB, 128),
                         lambda i: (jnp.maximum(i - n0, 0), 0)),
            full((64, 128)),
            full((256, 64)), full((1, 256)),
            full((128, 256)), full((1, 128)),
        ],
        out_specs=[
            pl.BlockSpec((_BB, 128), lambda i: (i, 0)),
            pl.BlockSpec((_BB, 1), lambda i: (i, 0)),
        ],
        out_shape=[
            jax.ShapeDtypeStruct((b, 128), jnp.float32),
            jax.ShapeDtypeStruct((b, 1), jnp.float32),
        ],
    )(hot0, hot1, cbw, dw1, db1.reshape(1, -1), dw2, db2.reshape(1, -1))
    return recon, kk


_SPLITS = (12288, 4096)


@jax.jit
def _run(x, ew1, eb1, ew2, eb2, gw1, gb1, gw2, gb2, cbw, dw1, db1, dw2, db2):
    hots, off = [], 0
    for cs in _SPLITS:
        hots.append(_chunk(x, off, cs, ew1, eb1, ew2, eb2,
                           gw1, gb1, gw2, gb2))
        off += cs
    recon, kk = _decode(hots[0], hots[1], cbw, dw1, db1, dw2, db2)
    hot = jnp.concatenate(hots)
    return recon, hot, kk


def kernel(x, enc_w1, enc_b1, enc_w2, enc_b2, gate_w1, gate_b1, gate_w2,
           gate_b2, cb_w, dec_w1, dec_b1, dec_w2, dec_b2):
    recon, hot, kk = _run(x, enc_w1, enc_b1, enc_w2, enc_b2, gate_w1, gate_b1,
                          gate_w2, gate_b2, cb_w, dec_w1, dec_b1, dec_w2,
                          dec_b2)
    return (recon, hot, jnp.float32(0.0), kk)


# speedup vs baseline: 1.2862x; 1.0377x over previous
import functools

import jax
import jax.numpy as jnp
from jax import lax
from jax.experimental import pallas as pl
from jax.experimental.pallas import tpu as pltpu
from jax.experimental.pallas import tpu_sc as plsc

_N = 16
_D = 16
_K = 8
_BB = 2048


def _lane_consts():
    l = jax.lax.broadcasted_iota(jnp.int32, (1, _N * _D), 1)
    return l // _D, l % _D


def _stage_a_body(x_ref, ew1, eb1, ew2, eb2, gw1, gb1, gw2, gb2,
                  khot_ref, fsel_ref):
    f32 = jnp.float32
    dot = lambda a, b: jax.lax.dot_general(
        a.astype(jnp.bfloat16), b.astype(jnp.bfloat16),
        (((1,), (1,)), ((), ())), preferred_element_type=f32)
    mml = lambda a, b: jax.lax.dot_general(
        a, b, (((1,), (0,)), ((), ())), preferred_element_type=f32)
    mmh = lambda a, b: jax.lax.dot_general(
        a, b, (((1,), (0,)), ((), ())), preferred_element_type=f32,
        precision=jax.lax.Precision.HIGHEST)

    x = x_ref[...]
    h1 = dot(x, ew1[...]) + eb1[...]
    h1 = h1 * jax.nn.sigmoid(h1)
    enc = dot(h1, ew2[...]) + eb2[...]
    g1 = dot(enc, gw1[...]) + gb1[...]
    g1 = g1 * jax.nn.sigmoid(g1)
    gate = dot(g1, gw2[...]) + gb2[...]

    seg_i, seg_j = _lane_consts()
    pi = jax.lax.broadcasted_iota(jnp.int32, (_N, _N * _D), 0)
    R_rep = (pi == seg_i).astype(f32)
    Mseg = (seg_i.T == jax.lax.broadcasted_iota(
        jnp.int32, (_N * _D, _N), 1)).astype(f32)

    gate_t = pltpu.repeat(gate, _N, 1)
    gate_r = mmh(gate, R_rep)
    tie = (seg_j < seg_i)
    beats = jnp.where((gate_t > gate_r) | ((gate_t == gate_r) & tie), 1.0, 0.0)
    rank16 = mml(beats, Mseg)

    S = (seg_i.T == seg_i).astype(f32)
    m = jnp.max(enc, axis=1, keepdims=True)
    e = jnp.exp(enc - m)
    sum_lane = mmh(e, S)
    khot_ref[...] = jnp.where(e * _D > sum_lane, 1.0, 0.0)

    lane128 = jax.lax.broadcasted_iota(jnp.int32, (1, _K * _N), 1)
    rconst = lane128 // _N
    iconst = (lane128 % _N).astype(f32)
    M128 = ((lane128.T // _N) == jax.lax.broadcasted_iota(
        jnp.int32, (_K * _N, _K), 1)).astype(f32)
    rank_rep = pltpu.repeat(rank16, _K, 1)
    selv = jnp.where(rank_rep == rconst.astype(f32), iconst, 0.0)
    sel8 = mml(selv, M128)
    row = (pl.program_id(0) * _BB
           + jax.lax.broadcasted_iota(jnp.int32, (_BB, 1), 0))
    fsel_ref[...] = (sel8 + 16.0 * row.astype(f32)).astype(jnp.int32)


def _stage_b_body(hot0_ref, hot1_ref, cbw, dw1, db1, dw2, db2,
                  recon_ref, kk_ref):
    f32 = jnp.float32
    dot = lambda a, b: jax.lax.dot_general(
        a.astype(jnp.bfloat16), b.astype(jnp.bfloat16),
        (((1,), (1,)), ((), ())), preferred_element_type=f32)
    n0 = _SPLITS[0] // _BB
    hot = jnp.where(pl.program_id(0) < n0, hot0_ref[...], hot1_ref[...])
    kk = jnp.clip(jnp.sum(hot, axis=1, keepdims=True), 1.0, 128.0)
    hot_n = hot / kk
    q = dot(hot_n, cbw[...])
    d1 = dot(q, dw1[...]) + db1[...]
    d1 = d1 * jax.nn.sigmoid(d1)
    recon_ref[...] = dot(d1, dw2[...]) + db2[...]
    kk_ref[...] = kk


def _sc_gather(table, idx2d, rows_total, b_per_w, nch, ch):
    mesh = plsc.VectorSubcoreMesh(core_axis_name="c", subcore_axis_name="s")

    @functools.partial(
        pl.kernel, mesh=mesh,
        compiler_params=pltpu.CompilerParams(use_tc_tiling_on_sc=False),
        out_type=jax.ShapeDtypeStruct((rows_total, _D), jnp.float32),
        scratch_types=[
            pltpu.VMEM((nch, ch), jnp.int32),
            pltpu.VMEM((b_per_w, _D), jnp.float32),
            pltpu.SemaphoreType.DMA,
        ],
    )
    def k(table_hbm, idx_hbm, out_hbm, idx_v, rows_v, sem):
        wid = lax.axis_index("s") * 2 + lax.axis_index("c")
        pltpu.sync_copy(idx_hbm.at[pl.ds(wid * nch, nch)], idx_v)
        copies = []
        for j in range(nch):
            copies.append(pltpu.async_copy(
                table_hbm.at[idx_v.at[j]],
                rows_v.at[pl.ds(j * ch, ch)], sem))
        for c in copies:
            c.wait()
        pltpu.sync_copy(rows_v, out_hbm.at[pl.ds(wid * b_per_w, b_per_w)])

    return k(table, idx2d)


def _chunk(x, off, b, ew1, eb1, ew2, eb2, gw1, gb1, gw2, gb2):
    grid = (b // _BB,)
    offb = off // _BB
    full = lambda shape: pl.BlockSpec(shape, lambda i: (0,) * len(shape))
    khot, fsel = pl.pallas_call(
        _stage_a_body,
        grid=grid,
        in_specs=[
            pl.BlockSpec((_BB, 128), lambda i: (i + offb, 0)),
            full((256, 128)), full((1, 256)),
            full((256, 256)), full((1, 256)),
            full((256, 256)), full((1, 256)),
            full((16, 256)), full((1, 16)),
        ],
        out_specs=[
            pl.BlockSpec((_BB, 256), lambda i: (i, 0)),
            pl.BlockSpec((_BB, _K), lambda i: (i, 0)),
        ],
        out_shape=[
            jax.ShapeDtypeStruct((b, 256), jnp.float32),
            jax.ShapeDtypeStruct((b, _K), jnp.int32),
        ],
    )(x, ew1, eb1.reshape(1, -1), ew2, eb2.reshape(1, -1), gw1,
      gb1.reshape(1, -1), gw2, gb2.reshape(1, -1))

    info = plsc.get_sparse_core_info()
    nw = info.num_cores * info.num_subcores
    rows_total = b * _K
    b_per_w = rows_total // nw
    ch = 128
    nch = b_per_w // ch
    table = khot.reshape(b * _N, _D)
    idx2d = fsel.reshape(rows_total // ch, ch)
    hot_rows = _sc_gather(table, idx2d, rows_total, b_per_w, nch, ch)
    return hot_rows.reshape(b, _K * _D)


def _decode(hot0, hot1, cbw, dw1, db1, dw2, db2):
    b = hot0.shape[0] + hot1.shape[0]
    grid = (b // _BB,)
    n0 = hot0.shape[0] // _BB
    full = lambda shape: pl.BlockSpec(shape, lambda i: (0,) * len(shape))
    recon, kk = pl.pallas_call(
        _stage_b_body,
        grid=grid,
        in_specs=[
            pl.BlockSpec((_BB, 128), lambda i: (jnp.minimum(i, n0 - 1), 0)),
            pl.BlockSpec((_BB, 128),
                         lambda i: (jnp.maximum(i - n0, 0), 0)),
            full((64, 128)),
            full((256, 64)), full((1, 256)),
            full((128, 256)), full((1, 128)),
        ],
        out_specs=[
            pl.BlockSpec((_BB, 128), lambda i: (i, 0)),
            pl.BlockSpec((_BB, 1), lambda i: (i, 0)),
        ],
        out_shape=[
            jax.ShapeDtypeStruct((b, 128), jnp.float32),
            jax.ShapeDtypeStruct((b, 1), jnp.float32),
        ],
    )(hot0, hot1, cbw, dw1, db1.reshape(1, -1), dw2, db2.reshape(1, -1))
    return recon, kk


_SPLITS = (12288, 4096)


@jax.jit
def _run(x, ew1, eb1, ew2, eb2, gw1, gb1, gw2, gb2, cbw, dw1, db1, dw2, db2):
    hots, off = [], 0
    for cs in _SPLITS:
        hots.append(_chunk(x, off, cs, ew1, eb1, ew2, eb2,
                           gw1, gb1, gw2, gb2))
        off += cs
    recon, kk = _decode(hots[0], hots[1], cbw, dw1, db1, dw2, db2)
    hot = jnp.concatenate(hots)
    return recon, hot, kk


def kernel(x, enc_w1, enc_b1, enc_w2, enc_b2, gate_w1, gate_b1, gate_w2,
           gate_b2, cb_w, dec_w1, dec_b1, dec_w2, dec_b2):
    recon, hot, kk = _run(x, enc_w1, enc_b1, enc_w2, enc_b2, gate_w1, gate_b1,
                          gate_w2, gate_b2, cb_w, dec_w1, dec_b1, dec_w2,
                          dec_b2)
    return (recon, hot, jnp.float32(0.0), kk)


# speedup vs baseline: 1.2930x; 1.0053x over previous
import functools

import jax
import jax.numpy as jnp
from jax import lax
from jax.experimental import pallas as pl
from jax.experimental.pallas import tpu as pltpu
from jax.experimental.pallas import tpu_sc as plsc

_N = 16
_D = 16
_K = 8
_BB = 2048


def _lane_consts():
    l = jax.lax.broadcasted_iota(jnp.int32, (1, _N * _D), 1)
    return l // _D, l % _D


def _stage_a_body(x_ref, ew1, eb1, ew2, eb2, gw1, gb1, gw2, gb2,
                  khot_ref, fsel_ref):
    f32 = jnp.float32
    dot = lambda a, b: jax.lax.dot_general(
        a.astype(jnp.bfloat16), b.astype(jnp.bfloat16),
        (((1,), (1,)), ((), ())), preferred_element_type=f32)
    mml = lambda a, b: jax.lax.dot_general(
        a, b, (((1,), (0,)), ((), ())), preferred_element_type=f32)
    mmh = lambda a, b: jax.lax.dot_general(
        a, b, (((1,), (0,)), ((), ())), preferred_element_type=f32,
        precision=jax.lax.Precision.HIGHEST)

    x = x_ref[...]
    h1 = dot(x, ew1[...]) + eb1[...]
    h1 = h1 * jax.nn.sigmoid(h1)
    enc = dot(h1, ew2[...]) + eb2[...]
    g1 = dot(enc, gw1[...]) + gb1[...]
    g1 = g1 * jax.nn.sigmoid(g1)
    gate = dot(g1, gw2[...]) + gb2[...]

    seg_i, seg_j = _lane_consts()
    pi = jax.lax.broadcasted_iota(jnp.int32, (_N, _N * _D), 0)
    R_rep = (pi == seg_i).astype(f32)
    Mseg = (seg_i.T == jax.lax.broadcasted_iota(
        jnp.int32, (_N * _D, _N), 1)).astype(f32)

    gate_t = pltpu.repeat(gate, _N, 1)
    gate_r = mmh(gate, R_rep)
    tie = (seg_j < seg_i)
    beats = jnp.where((gate_t > gate_r) | ((gate_t == gate_r) & tie), 1.0, 0.0)
    rank16 = mml(beats, Mseg)

    S = (seg_i.T == seg_i).astype(f32)
    m = jnp.max(enc, axis=1, keepdims=True)
    e = jnp.exp(enc - m)
    sum_lane = mmh(e, S)
    khot_ref[...] = jnp.where(e * _D > sum_lane, 1.0, 0.0)

    lane128 = jax.lax.broadcasted_iota(jnp.int32, (1, _K * _N), 1)
    rconst = lane128 // _N
    iconst = (lane128 % _N).astype(f32)
    M128 = ((lane128.T // _N) == jax.lax.broadcasted_iota(
        jnp.int32, (_K * _N, _K), 1)).astype(f32)
    rank_rep = pltpu.repeat(rank16, _K, 1)
    selv = jnp.where(rank_rep == rconst.astype(f32), iconst, 0.0)
    sel8 = mml(selv, M128)
    row = (pl.program_id(0) * _BB
           + jax.lax.broadcasted_iota(jnp.int32, (_BB, 1), 0))
    fsel_ref[...] = (sel8 + 16.0 * row.astype(f32)).astype(jnp.int32)


def _stage_b_body(hot0_ref, hot1_ref, cbw, dw1, db1, dw2, db2,
                  recon_ref, kk_ref):
    f32 = jnp.float32
    dot = lambda a, b: jax.lax.dot_general(
        a.astype(jnp.bfloat16), b.astype(jnp.bfloat16),
        (((1,), (1,)), ((), ())), preferred_element_type=f32)
    n0 = _SPLITS[0] // _BB
    hot = jnp.where(pl.program_id(0) < n0, hot0_ref[...], hot1_ref[...])
    kk = jnp.clip(jnp.sum(hot, axis=1, keepdims=True), 1.0, 128.0)
    hot_n = hot / kk
    q = dot(hot_n, cbw[...])
    d1 = dot(q, dw1[...]) + db1[...]
    d1 = d1 * jax.nn.sigmoid(d1)
    recon_ref[...] = dot(d1, dw2[...]) + db2[...]
    kk_ref[...] = kk


def _sc_gather(table, idx2d, rows_total, b_per_w, nch, ch):
    mesh = plsc.VectorSubcoreMesh(core_axis_name="c", subcore_axis_name="s")

    @functools.partial(
        pl.kernel, mesh=mesh,
        compiler_params=pltpu.CompilerParams(use_tc_tiling_on_sc=False),
        out_type=jax.ShapeDtypeStruct((rows_total, _D), jnp.float32),
        scratch_types=[
            pltpu.VMEM((nch, ch), jnp.int32),
            pltpu.VMEM((b_per_w, _D), jnp.float32),
            pltpu.SemaphoreType.DMA,
        ],
    )
    def k(table_hbm, idx_hbm, out_hbm, idx_v, rows_v, sem):
        wid = lax.axis_index("s") * 2 + lax.axis_index("c")
        pltpu.sync_copy(idx_hbm.at[pl.ds(wid * nch, nch)], idx_v)
        copies = []
        for j in range(nch):
            copies.append(pltpu.async_copy(
                table_hbm.at[idx_v.at[j]],
                rows_v.at[pl.ds(j * ch, ch)], sem))
        for c in copies:
            c.wait()
        pltpu.sync_copy(rows_v, out_hbm.at[pl.ds(wid * b_per_w, b_per_w)])

    return k(table, idx2d)


def _chunk(x, off, b, ew1, eb1, ew2, eb2, gw1, gb1, gw2, gb2):
    grid = (b // _BB,)
    offb = off // _BB
    full = lambda shape: pl.BlockSpec(shape, lambda i: (0,) * len(shape))
    khot, fsel = pl.pallas_call(
        _stage_a_body,
        grid=grid,
        in_specs=[
            pl.BlockSpec((_BB, 128), lambda i: (i + offb, 0)),
            full((256, 128)), full((1, 256)),
            full((256, 256)), full((1, 256)),
            full((256, 256)), full((1, 256)),
            full((16, 256)), full((1, 16)),
        ],
        out_specs=[
            pl.BlockSpec((_BB, 256), lambda i: (i, 0)),
            pl.BlockSpec((_BB, _K), lambda i: (i, 0)),
        ],
        out_shape=[
            jax.ShapeDtypeStruct((b, 256), jnp.float32),
            jax.ShapeDtypeStruct((b, _K), jnp.int32),
        ],
    )(x, ew1, eb1.reshape(1, -1), ew2, eb2.reshape(1, -1), gw1,
      gb1.reshape(1, -1), gw2, gb2.reshape(1, -1))

    info = plsc.get_sparse_core_info()
    nw = info.num_cores * info.num_subcores
    rows_total = b * _K
    b_per_w = rows_total // nw
    ch = 128
    nch = b_per_w // ch
    table = khot.reshape(b * _N, _D)
    idx2d = fsel.reshape(rows_total // ch, ch)
    hot_rows = _sc_gather(table, idx2d, rows_total, b_per_w, nch, ch)
    return hot_rows.reshape(b, _K * _D)


def _decode(hot0, hot1, cbw, dw1, db1, dw2, db2):
    b = hot0.shape[0] + hot1.shape[0]
    grid = (b // _BB,)
    n0 = hot0.shape[0] // _BB
    full = lambda shape: pl.BlockSpec(shape, lambda i: (0,) * len(shape))
    recon, kk = pl.pallas_call(
        _stage_b_body,
        grid=grid,
        in_specs=[
            pl.BlockSpec((_BB, 128), lambda i: (jnp.minimum(i, n0 - 1), 0)),
            pl.BlockSpec((_BB, 128),
                         lambda i: (jnp.maximum(i - n0, 0), 0)),
            full((64, 128)),
            full((256, 64)), full((1, 256)),
            full((128, 256)), full((1, 128)),
        ],
        out_specs=[
            pl.BlockSpec((_BB, 128), lambda i: (i, 0)),
            pl.BlockSpec((_BB, 1), lambda i: (i, 0)),
        ],
        out_shape=[
            jax.ShapeDtypeStruct((b, 128), jnp.float32),
            jax.ShapeDtypeStruct((b, 1), jnp.float32),
        ],
    )(hot0, hot1, cbw, dw1, db1.reshape(1, -1), dw2, db2.reshape(1, -1))
    return recon, kk


_SPLITS = (14336, 2048)


@jax.jit
def _run(x, ew1, eb1, ew2, eb2, gw1, gb1, gw2, gb2, cbw, dw1, db1, dw2, db2):
    hots, off = [], 0
    for cs in _SPLITS:
        hots.append(_chunk(x, off, cs, ew1, eb1, ew2, eb2,
                           gw1, gb1, gw2, gb2))
        off += cs
    recon, kk = _decode(hots[0], hots[1], cbw, dw1, db1, dw2, db2)
    hot = jnp.concatenate(hots)
    return recon, hot, kk


def kernel(x, enc_w1, enc_b1, enc_w2, enc_b2, gate_w1, gate_b1, gate_w2,
           gate_b2, cb_w, dec_w1, dec_b1, dec_w2, dec_b2):
    recon, hot, kk = _run(x, enc_w1, enc_b1, enc_w2, enc_b2, gate_w1, gate_b1,
                          gate_w2, gate_b2, cb_w, dec_w1, dec_b1, dec_w2,
                          dec_b2)
    return (recon, hot, jnp.float32(0.0), kk)


# speedup vs baseline: 1.3000x; 1.0054x over previous
import functools

import jax
import jax.numpy as jnp
from jax import lax
from jax.experimental import pallas as pl
from jax.experimental.pallas import tpu as pltpu
from jax.experimental.pallas import tpu_sc as plsc

_N = 16
_D = 16
_K = 8
_BB = 2048


def _lane_consts():
    l = jax.lax.broadcasted_iota(jnp.int32, (1, _N * _D), 1)
    return l // _D, l % _D


def _stage_a_body(x_ref, ew1, eb1, ew2, eb2, gw1, gb1, gw2, gb2,
                  khot_ref, fsel_ref):
    f32 = jnp.float32
    dot = lambda a, b: jax.lax.dot_general(
        a.astype(jnp.bfloat16), b.astype(jnp.bfloat16),
        (((1,), (1,)), ((), ())), preferred_element_type=f32)
    mml = lambda a, b: jax.lax.dot_general(
        a, b, (((1,), (0,)), ((), ())), preferred_element_type=f32)
    mmh = lambda a, b: jax.lax.dot_general(
        a, b, (((1,), (0,)), ((), ())), preferred_element_type=f32,
        precision=jax.lax.Precision.HIGHEST)

    x = x_ref[...]
    h1 = dot(x, ew1[...]) + eb1[...]
    h1 = h1 * jax.nn.sigmoid(h1)
    enc = dot(h1, ew2[...]) + eb2[...]
    g1 = dot(enc, gw1[...]) + gb1[...]
    g1 = g1 * jax.nn.sigmoid(g1)
    gate = dot(g1, gw2[...]) + gb2[...]

    seg_i, seg_j = _lane_consts()
    pi = jax.lax.broadcasted_iota(jnp.int32, (_N, _N * _D), 0)
    R_rep = (pi == seg_i).astype(f32)
    Mseg = (seg_i.T == jax.lax.broadcasted_iota(
        jnp.int32, (_N * _D, _N), 1)).astype(f32)

    gate_t = pltpu.repeat(gate, _N, 1)
    gate_r = mmh(gate, R_rep)
    tie = (seg_j < seg_i)
    beats = jnp.where((gate_t > gate_r) | ((gate_t == gate_r) & tie), 1.0, 0.0)
    rank16 = mml(beats, Mseg)

    S = (seg_i.T == seg_i).astype(f32)
    m = jnp.max(enc, axis=1, keepdims=True)
    e = jnp.exp(enc - m)
    sum_lane = mmh(e, S)
    khot_ref[...] = jnp.where(e * _D > sum_lane, 1.0, 0.0)

    lane128 = jax.lax.broadcasted_iota(jnp.int32, (1, _K * _N), 1)
    rconst = lane128 // _N
    iconst = (lane128 % _N).astype(f32)
    M128 = ((lane128.T // _N) == jax.lax.broadcasted_iota(
        jnp.int32, (_K * _N, _K), 1)).astype(f32)
    rank_rep = pltpu.repeat(rank16, _K, 1)
    selv = jnp.where(rank_rep == rconst.astype(f32), iconst, 0.0)
    sel8 = mml(selv, M128)
    row = (pl.program_id(0) * _BB
           + jax.lax.broadcasted_iota(jnp.int32, (_BB, 1), 0))
    fsel_ref[...] = (sel8 + 16.0 * row.astype(f32)).astype(jnp.int32)


def _stage_b_body(hot_ref, cbw, dw1, db1, dw2, db2, recon_ref, kk_ref):
    f32 = jnp.float32
    dot = lambda a, b: jax.lax.dot_general(
        a.astype(jnp.bfloat16), b.astype(jnp.bfloat16),
        (((1,), (1,)), ((), ())), preferred_element_type=f32)
    hot = hot_ref[...]
    kk = jnp.clip(jnp.sum(hot, axis=1, keepdims=True), 1.0, 128.0)
    hot_n = hot / kk
    q = dot(hot_n, cbw[...])
    d1 = dot(q, dw1[...]) + db1[...]
    d1 = d1 * jax.nn.sigmoid(d1)
    recon_ref[...] = dot(d1, dw2[...]) + db2[...]
    kk_ref[...] = kk


def _sc_gather(table, idx2d, rows_total, b_per_w, nch, ch):
    mesh = plsc.VectorSubcoreMesh(core_axis_name="c", subcore_axis_name="s")

    @functools.partial(
        pl.kernel, mesh=mesh,
        compiler_params=pltpu.CompilerParams(use_tc_tiling_on_sc=False),
        out_type=jax.ShapeDtypeStruct((rows_total, _D), jnp.float32),
        scratch_types=[
            pltpu.VMEM((nch, ch), jnp.int32),
            pltpu.VMEM((b_per_w, _D), jnp.float32),
            pltpu.SemaphoreType.DMA,
        ],
    )
    def k(table_hbm, idx_hbm, out_hbm, idx_v, rows_v, sem):
        wid = lax.axis_index("s") * 2 + lax.axis_index("c")
        pltpu.sync_copy(idx_hbm.at[pl.ds(wid * nch, nch)], idx_v)
        copies = []
        for j in range(nch):
            copies.append(pltpu.async_copy(
                table_hbm.at[idx_v.at[j]],
                rows_v.at[pl.ds(j * ch, ch)], sem))
        for c in copies:
            c.wait()
        pltpu.sync_copy(rows_v, out_hbm.at[pl.ds(wid * b_per_w, b_per_w)])

    return k(table, idx2d)


def _chunk(x, off, b, ew1, eb1, ew2, eb2, gw1, gb1, gw2, gb2):
    grid = (b // _BB,)
    offb = off // _BB
    full = lambda shape: pl.BlockSpec(shape, lambda i: (0,) * len(shape))
    khot, fsel = pl.pallas_call(
        _stage_a_body,
        grid=grid,
        in_specs=[
            pl.BlockSpec((_BB, 128), lambda i: (i + offb, 0)),
            full((256, 128)), full((1, 256)),
            full((256, 256)), full((1, 256)),
            full((256, 256)), full((1, 256)),
            full((16, 256)), full((1, 16)),
        ],
        out_specs=[
            pl.BlockSpec((_BB, 256), lambda i: (i, 0)),
            pl.BlockSpec((_BB, _K), lambda i: (i, 0)),
        ],
        out_shape=[
            jax.ShapeDtypeStruct((b, 256), jnp.float32),
            jax.ShapeDtypeStruct((b, _K), jnp.int32),
        ],
    )(x, ew1, eb1.reshape(1, -1), ew2, eb2.reshape(1, -1), gw1,
      gb1.reshape(1, -1), gw2, gb2.reshape(1, -1))

    info = plsc.get_sparse_core_info()
    nw = info.num_cores * info.num_subcores
    rows_total = b * _K
    b_per_w = rows_total // nw
    ch = 128
    nch = b_per_w // ch
    table = khot.reshape(b * _N, _D)
    idx2d = fsel.reshape(rows_total // ch, ch)
    hot_rows = _sc_gather(table, idx2d, rows_total, b_per_w, nch, ch)
    return hot_rows.reshape(b, _K * _D)


def _decode(hot, cbw, dw1, db1, dw2, db2):
    b = hot.shape[0]
    grid = (b // _BB,)
    full = lambda shape: pl.BlockSpec(shape, lambda i: (0,) * len(shape))
    recon, kk = pl.pallas_call(
        _stage_b_body,
        grid=grid,
        in_specs=[
            pl.BlockSpec((_BB, 128), lambda i: (i, 0)),
            full((64, 128)),
            full((256, 64)), full((1, 256)),
            full((128, 256)), full((1, 128)),
        ],
        out_specs=[
            pl.BlockSpec((_BB, 128), lambda i: (i, 0)),
            pl.BlockSpec((_BB, 1), lambda i: (i, 0)),
        ],
        out_shape=[
            jax.ShapeDtypeStruct((b, 128), jnp.float32),
            jax.ShapeDtypeStruct((b, 1), jnp.float32),
        ],
    )(hot, cbw, dw1, db1.reshape(1, -1), dw2, db2.reshape(1, -1))
    return recon, kk


_SPLITS = (14336, 2048)


@jax.jit
def _run(x, ew1, eb1, ew2, eb2, gw1, gb1, gw2, gb2, cbw, dw1, db1, dw2, db2):
    hots, off = [], 0
    for cs in _SPLITS:
        hots.append(_chunk(x, off, cs, ew1, eb1, ew2, eb2,
                           gw1, gb1, gw2, gb2))
        off += cs
    hot = jnp.concatenate(hots) if len(hots) > 1 else hots[0]
    recon, kk = _decode(hot, cbw, dw1, db1, dw2, db2)
    return recon, hot, kk


def kernel(x, enc_w1, enc_b1, enc_w2, enc_b2, gate_w1, gate_b1, gate_w2,
           gate_b2, cb_w, dec_w1, dec_b1, dec_w2, dec_b2):
    recon, hot, kk = _run(x, enc_w1, enc_b1, enc_w2, enc_b2, gate_w1, gate_b1,
                          gate_w2, gate_b2, cb_w, dec_w1, dec_b1, dec_w2,
                          dec_b2)
    return (recon, hot, jnp.float32(0.0), kk)
